# baseline (speedup 1.0000x reference)
import functools

import jax
import jax.numpy as jnp
from jax import lax
from jax.experimental import pallas as pl
from jax.experimental.pallas import tpu as pltpu
from jax.experimental.pallas import tpu_sc as plsc

_N = 10000
_E = 320000
_D = 128
_NC = 2
_NS = 16
_NW = _NC * _NS
_EW = _E // _NW
_C = 80
_NCH = _EW // _C
_RPT = 624
_TAIL = _N - _RPT * _NS


@functools.cache
def _make_agg_kernel():
  mesh = plsc.VectorSubcoreMesh(core_axis_name="c", subcore_axis_name="s",
                                num_cores=_NC, num_subcores=_NS)

  @functools.partial(
      pl.kernel,
      mesh=mesh,
      out_type=jax.ShapeDtypeStruct((_NC, _N, _D), jnp.float32),
      scratch_types=[
          pltpu.VMEM((_EW,), jnp.int32),
          pltpu.VMEM((_NCH, _C), jnp.int32),
          pltpu.VMEM((_C, _D), jnp.float32),
          pltpu.VMEM((_C, _D), jnp.float32),
          pltpu.VMEM_SHARED((_N, _D), jnp.float32),
          pltpu.SemaphoreType.DMA,
          pltpu.SemaphoreType.DMA,
          pltpu.SemaphoreType.DMA,
      ],
  )
  def agg_kernel(x_hbm, src_hbm, dst_hbm, zeros_hbm, out_hbm,
                 src_v, dst_v, rows0, rows1, agg_sh, sem0, sem1, zsem):
    c = lax.axis_index("c")
    s = lax.axis_index("s")
    wid = c * _NS + s
    @pl.when(c == 0)
    def _():
      pltpu.async_copy(
          x_hbm.at[pl.ds(s * _RPT, _RPT)],
          agg_sh.at[pl.ds(s * _RPT, _RPT)], zsem)

    @pl.when(c != 0)
    def _():
      pltpu.async_copy(
          zeros_hbm.at[pl.ds(s * _RPT, _RPT)],
          agg_sh.at[pl.ds(s * _RPT, _RPT)], zsem)

    src_cp = pltpu.async_copy(src_hbm.at[pl.ds(wid * _EW, _EW)], src_v, sem0)
    dst_cp = pltpu.async_copy(dst_hbm.at[wid], dst_v, sem1)
    src_cp.wait()
    dst_cp.wait()
    pltpu.make_async_copy(
        zeros_hbm.at[pl.ds(s * _RPT, _RPT)],
        agg_sh.at[pl.ds(s * _RPT, _RPT)], zsem).wait()

    @pl.when(s == _NS - 1)
    def _():
      @pl.when(c == 0)
      def _():
        pltpu.sync_copy(
            x_hbm.at[pl.ds(_RPT * _NS, _TAIL)],
            agg_sh.at[pl.ds(_RPT * _NS, _TAIL)])

      @pl.when(c != 0)
      def _():
        pltpu.sync_copy(
            zeros_hbm.at[pl.ds(_RPT * _NS, _TAIL)],
            agg_sh.at[pl.ds(_RPT * _NS, _TAIL)])

    plsc.subcore_barrier()

    def src_slice(j):
      return src_v.at[pl.ds(pl.multiple_of(j * _C, 8), _C)]

    pltpu.async_copy(x_hbm.at[src_slice(0)], rows0, sem0)
    pltpu.async_copy(x_hbm.at[src_slice(1)], rows1, sem1)

    def body(i, carry):
      for b, rows, sem in ((0, rows0, sem0), (1, rows1, sem1)):
        j = 2 * i + b
        pltpu.make_async_copy(x_hbm.at[src_slice(j)], rows, sem).wait()
        pltpu.sync_copy(rows, agg_sh.at[dst_v.at[j]], add=True)
        nxt = j + 2

        @pl.when(nxt < _NCH)
        def _(rows=rows, sem=sem, nxt=nxt):
          pltpu.async_copy(x_hbm.at[src_slice(nxt)], rows, sem)

      return carry

    lax.fori_loop(0, _NCH // 2, body, 0, unroll=False)
    if _NCH % 2:
      j = _NCH - 1
      pltpu.make_async_copy(x_hbm.at[src_slice(j)], rows0, sem0).wait()
      pltpu.sync_copy(rows0, agg_sh.at[dst_v.at[j]], add=True)
    plsc.subcore_barrier()
    pltpu.sync_copy(
        agg_sh.at[pl.ds(s * _RPT, _RPT)],
        out_hbm.at[c, pl.ds(s * _RPT, _RPT)])

    @pl.when(s == _NS - 1)
    def _():
      pltpu.sync_copy(
          agg_sh.at[pl.ds(_RPT * _NS, _TAIL)],
          out_hbm.at[c, pl.ds(_RPT * _NS, _TAIL)])

  return agg_kernel


_B = 2000
_K = _N // _B


def _dense_body(last, agg_ref, w1_ref, b1_ref, g_ref, bt_ref,
                w2_ref, b2_ref, og_ref, ob_ref, o_ref,
                t_ref, s1_ref, q1_ref, s2_ref, q2_ref):
  i = pl.program_id(0)

  @pl.when(i == 0)
  def _():
    s1_ref[...] = jnp.zeros_like(s1_ref)
    q1_ref[...] = jnp.zeros_like(q1_ref)
    s2_ref[...] = jnp.zeros_like(s2_ref)
    q2_ref[...] = jnp.zeros_like(q2_ref)

  @pl.when(i < _K)
  def _():
    blk = agg_ref[0] + agg_ref[1]
    t = jnp.dot(blk, w1_ref[...],
                preferred_element_type=jnp.float32) + b1_ref[...]
    t_ref[pl.ds(i * _B, _B), :] = t
    s1_ref[...] += jnp.sum(t, axis=0, keepdims=True)
    q1_ref[...] += jnp.sum(t * t, axis=0, keepdims=True)

  @pl.when((i >= _K) & (i < 2 * _K))
  def _():
    ib = i - _K
    t = t_ref[pl.ds(ib * _B, _B), :]
    m = s1_ref[...] * (1.0 / _N)
    var = q1_ref[...] * (1.0 / _N) - m * m
    r = jnp.maximum((t - m) * (g_ref[...] * lax.rsqrt(var + 1e-5))
                    + bt_ref[...], 0.0)
    v = jnp.dot(r, w2_ref[...],
                preferred_element_type=jnp.float32) + b2_ref[...]
    if last:
      nrm = jnp.sqrt(jnp.sum(v * v, axis=1, keepdims=True))
      o_ref[...] = v / jnp.maximum(nrm, 1e-12)
    else:
      t_ref[pl.ds(ib * _B, _B), :] = v
      s2_ref[...] += jnp.sum(v, axis=0, keepdims=True)
      q2_ref[...] += jnp.sum(v * v, axis=0, keepdims=True)

  if not last:
    @pl.when(i >= 2 * _K)
    def _():
      ic = i - 2 * _K
      v = t_ref[pl.ds(ic * _B, _B), :]
      m2 = s2_ref[...] * (1.0 / _N)
      var2 = q2_ref[...] * (1.0 / _N) - m2 * m2
      o_ref[...] = jnp.maximum(
          (v - m2) * (og_ref[...] * lax.rsqrt(var2 + 1e-5)) + ob_ref[...],
          0.0)


def _dense_layer(agg, w1, b1, g, bt, w2, b2, og, ob, last):
  nsteps = (2 if last else 3) * _K
  ofs = (1 if last else 2) * _K
  full = lambda i: (0, 0)
  return pl.pallas_call(
      functools.partial(_dense_body, last),
      grid=(nsteps,),
      in_specs=[
          pl.BlockSpec((_NC, _B, _D),
                       lambda i: (0, jnp.minimum(i, _K - 1), 0)),
          pl.BlockSpec((_D, _D), full),
          pl.BlockSpec((1, _D), full),
          pl.BlockSpec((1, _D), full),
          pl.BlockSpec((1, _D), full),
          pl.BlockSpec((_D, _D), full),
          pl.BlockSpec((1, _D), full),
          pl.BlockSpec((1, _D), full),
          pl.BlockSpec((1, _D), full),
      ],
      out_specs=pl.BlockSpec((_B, _D),
                             lambda i: (jnp.maximum(i - ofs, 0), 0)),
      out_shape=jax.ShapeDtypeStruct((_N, _D), jnp.float32),
      scratch_shapes=[
          pltpu.VMEM((_N, _D), jnp.float32),
          pltpu.VMEM((1, _D), jnp.float32),
          pltpu.VMEM((1, _D), jnp.float32),
          pltpu.VMEM((1, _D), jnp.float32),
          pltpu.VMEM((1, _D), jnp.float32),
      ],
  )(agg, w1, b1.reshape(1, _D), g.reshape(1, _D), bt.reshape(1, _D),
    w2, b2.reshape(1, _D), og.reshape(1, _D), ob.reshape(1, _D))


def kernel(x, edge_index,
           W1_0, b1_0, g_0, bt_0, W2_0, b2_0,
           W1_1, b1_1, g_1, bt_1, W2_1, b2_1,
           W1_2, b1_2, g_2, bt_2, W2_2, b2_2,
           og_0, ob_0, og_1, ob_1):
  src = edge_index[0]
  dst = edge_index[1].reshape(_NW, _NCH, _C)
  zeros = jnp.zeros((_N, _D), jnp.float32)
  params = [
      (W1_0, b1_0, g_0, bt_0, W2_0, b2_0, og_0, ob_0),
      (W1_1, b1_1, g_1, bt_1, W2_1, b2_1, og_1, ob_1),
      (W1_2, b1_2, g_2, bt_2, W2_2, b2_2, og_0, ob_0),
  ]
  for l in range(3):
    w1, b1, g, bt, w2, b2, og, ob = params[l]
    agg = _make_agg_kernel()(x, src, dst, zeros)
    x = _dense_layer(agg, w1, b1, g, bt, w2, b2, og, ob, last=(l == 2))
  return x


# speedup vs baseline: 1.0081x; 1.0081x over previous
import functools

import jax
import jax.numpy as jnp
from jax import lax
from jax.experimental import pallas as pl
from jax.experimental.pallas import tpu as pltpu
from jax.experimental.pallas import tpu_sc as plsc

_N = 10000
_E = 320000
_D = 128
_NC = 2
_NS = 16
_NW = _NC * _NS
_EW = _E // _NW
_C = 80
_NCH = _EW // _C
_RPT = 624
_TAIL = _N - _RPT * _NS


@functools.cache
def _make_agg_kernel():
  mesh = plsc.VectorSubcoreMesh(core_axis_name="c", subcore_axis_name="s",
                                num_cores=_NC, num_subcores=_NS)

  @functools.partial(
      pl.kernel,
      mesh=mesh,
      out_type=jax.ShapeDtypeStruct((_NC, _N, _D), jnp.float32),
      scratch_types=[
          pltpu.VMEM((_EW,), jnp.int32),
          pltpu.VMEM((_EW,), jnp.int32),
          pltpu.VMEM((_C, _D), jnp.float32),
          pltpu.VMEM((_C, _D), jnp.float32),
          pltpu.VMEM_SHARED((_N, _D), jnp.float32),
          pltpu.SemaphoreType.DMA,
          pltpu.SemaphoreType.DMA,
          pltpu.SemaphoreType.DMA,
      ],
  )
  def agg_kernel(x_hbm, src_hbm, dst_hbm, zeros_hbm, out_hbm,
                 src_v, dst_v, rows0, rows1, agg_sh, sem0, sem1, zsem):
    c = lax.axis_index("c")
    s = lax.axis_index("s")
    wid = c * _NS + s
    @pl.when(c == 0)
    def _():
      pltpu.async_copy(
          x_hbm.at[pl.ds(s * _RPT, _RPT)],
          agg_sh.at[pl.ds(s * _RPT, _RPT)], zsem)

    @pl.when(c != 0)
    def _():
      pltpu.async_copy(
          zeros_hbm.at[pl.ds(s * _RPT, _RPT)],
          agg_sh.at[pl.ds(s * _RPT, _RPT)], zsem)

    src_cp = pltpu.async_copy(src_hbm.at[pl.ds(wid * _EW, _EW)], src_v, sem0)
    dst_cp = pltpu.async_copy(dst_hbm.at[pl.ds(wid * _EW, _EW)], dst_v, sem1)
    src_cp.wait()
    dst_cp.wait()
    pltpu.make_async_copy(
        zeros_hbm.at[pl.ds(s * _RPT, _RPT)],
        agg_sh.at[pl.ds(s * _RPT, _RPT)], zsem).wait()

    @pl.when(s == _NS - 1)
    def _():
      @pl.when(c == 0)
      def _():
        pltpu.sync_copy(
            x_hbm.at[pl.ds(_RPT * _NS, _TAIL)],
            agg_sh.at[pl.ds(_RPT * _NS, _TAIL)])

      @pl.when(c != 0)
      def _():
        pltpu.sync_copy(
            zeros_hbm.at[pl.ds(_RPT * _NS, _TAIL)],
            agg_sh.at[pl.ds(_RPT * _NS, _TAIL)])

    plsc.subcore_barrier()

    def src_slice(j):
      return src_v.at[pl.ds(pl.multiple_of(j * _C, 8), _C)]

    def dst_slice(j):
      return dst_v.at[pl.ds(pl.multiple_of(j * _C, 8), _C)]

    pltpu.async_copy(x_hbm.at[src_slice(0)], rows0, sem0)
    pltpu.async_copy(x_hbm.at[src_slice(1)], rows1, sem1)

    def body(i, carry):
      for b, rows, sem in ((0, rows0, sem0), (1, rows1, sem1)):
        j = 2 * i + b
        pltpu.make_async_copy(x_hbm.at[src_slice(j)], rows, sem).wait()
        pltpu.sync_copy(rows, agg_sh.at[dst_slice(j)], add=True)
        nxt = j + 2

        @pl.when(nxt < _NCH)
        def _(rows=rows, sem=sem, nxt=nxt):
          pltpu.async_copy(x_hbm.at[src_slice(nxt)], rows, sem)

      return carry

    lax.fori_loop(0, _NCH // 2, body, 0, unroll=False)
    if _NCH % 2:
      j = _NCH - 1
      pltpu.make_async_copy(x_hbm.at[src_slice(j)], rows0, sem0).wait()
      pltpu.sync_copy(rows0, agg_sh.at[dst_slice(j)], add=True)
    plsc.subcore_barrier()
    pltpu.sync_copy(
        agg_sh.at[pl.ds(s * _RPT, _RPT)],
        out_hbm.at[c, pl.ds(s * _RPT, _RPT)])

    @pl.when(s == _NS - 1)
    def _():
      pltpu.sync_copy(
          agg_sh.at[pl.ds(_RPT * _NS, _TAIL)],
          out_hbm.at[c, pl.ds(_RPT * _NS, _TAIL)])

  return agg_kernel


_B = 2000
_K = _N // _B


def _dense_body(last, agg_ref, w1_ref, b1_ref, g_ref, bt_ref,
                w2_ref, b2_ref, og_ref, ob_ref, o_ref,
                t_ref, s1_ref, q1_ref, s2_ref, q2_ref):
  i = pl.program_id(0)

  @pl.when(i == 0)
  def _():
    s1_ref[...] = jnp.zeros_like(s1_ref)
    q1_ref[...] = jnp.zeros_like(q1_ref)
    s2_ref[...] = jnp.zeros_like(s2_ref)
    q2_ref[...] = jnp.zeros_like(q2_ref)

  @pl.when(i < _K)
  def _():
    blk = agg_ref[0] + agg_ref[1]
    t = jnp.dot(blk, w1_ref[...],
                preferred_element_type=jnp.float32) + b1_ref[...]
    t_ref[pl.ds(i * _B, _B), :] = t
    s1_ref[...] += jnp.sum(t, axis=0, keepdims=True)
    q1_ref[...] += jnp.sum(t * t, axis=0, keepdims=True)

  @pl.when((i >= _K) & (i < 2 * _K))
  def _():
    ib = i - _K
    t = t_ref[pl.ds(ib * _B, _B), :]
    m = s1_ref[...] * (1.0 / _N)
    var = q1_ref[...] * (1.0 / _N) - m * m
    r = jnp.maximum((t - m) * (g_ref[...] * lax.rsqrt(var + 1e-5))
                    + bt_ref[...], 0.0)
    v = jnp.dot(r, w2_ref[...],
                preferred_element_type=jnp.float32) + b2_ref[...]
    if last:
      nrm = jnp.sqrt(jnp.sum(v * v, axis=1, keepdims=True))
      o_ref[...] = v / jnp.maximum(nrm, 1e-12)
    else:
      t_ref[pl.ds(ib * _B, _B), :] = v
      s2_ref[...] += jnp.sum(v, axis=0, keepdims=True)
      q2_ref[...] += jnp.sum(v * v, axis=0, keepdims=True)

  if not last:
    @pl.when(i >= 2 * _K)
    def _():
      ic = i - 2 * _K
      v = t_ref[pl.ds(ic * _B, _B), :]
      m2 = s2_ref[...] * (1.0 / _N)
      var2 = q2_ref[...] * (1.0 / _N) - m2 * m2
      o_ref[...] = jnp.maximum(
          (v - m2) * (og_ref[...] * lax.rsqrt(var2 + 1e-5)) + ob_ref[...],
          0.0)


def _dense_layer(agg, w1, b1, g, bt, w2, b2, og, ob, last):
  nsteps = (2 if last else 3) * _K
  ofs = (1 if last else 2) * _K
  full = lambda i: (0, 0)
  return pl.pallas_call(
      functools.partial(_dense_body, last),
      grid=(nsteps,),
      in_specs=[
          pl.BlockSpec((_NC, _B, _D),
                       lambda i: (0, jnp.minimum(i, _K - 1), 0)),
          pl.BlockSpec((_D, _D), full),
          pl.BlockSpec((1, _D), full),
          pl.BlockSpec((1, _D), full),
          pl.BlockSpec((1, _D), full),
          pl.BlockSpec((_D, _D), full),
          pl.BlockSpec((1, _D), full),
          pl.BlockSpec((1, _D), full),
          pl.BlockSpec((1, _D), full),
      ],
      out_specs=pl.BlockSpec((_B, _D),
                             lambda i: (jnp.maximum(i - ofs, 0), 0)),
      out_shape=jax.ShapeDtypeStruct((_N, _D), jnp.float32),
      scratch_shapes=[
          pltpu.VMEM((_N, _D), jnp.float32),
          pltpu.VMEM((1, _D), jnp.float32),
          pltpu.VMEM((1, _D), jnp.float32),
          pltpu.VMEM((1, _D), jnp.float32),
          pltpu.VMEM((1, _D), jnp.float32),
      ],
  )(agg, w1, b1.reshape(1, _D), g.reshape(1, _D), bt.reshape(1, _D),
    w2, b2.reshape(1, _D), og.reshape(1, _D), ob.reshape(1, _D))


def kernel(x, edge_index,
           W1_0, b1_0, g_0, bt_0, W2_0, b2_0,
           W1_1, b1_1, g_1, bt_1, W2_1, b2_1,
           W1_2, b1_2, g_2, bt_2, W2_2, b2_2,
           og_0, ob_0, og_1, ob_1):
  src = edge_index[0]
  dst = edge_index[1]
  zeros = jnp.zeros((_N, _D), jnp.float32)
  params = [
      (W1_0, b1_0, g_0, bt_0, W2_0, b2_0, og_0, ob_0),
      (W1_1, b1_1, g_1, bt_1, W2_1, b2_1, og_1, ob_1),
      (W1_2, b1_2, g_2, bt_2, W2_2, b2_2, og_0, ob_0),
  ]
  for l in range(3):
    w1, b1, g, bt, w2, b2, og, ob = params[l]
    agg = _make_agg_kernel()(x, src, dst, zeros)
    x = _dense_layer(agg, w1, b1, g, bt, w2, b2, og, ob, last=(l == 2))
  return x


# speedup vs baseline: 1.0905x; 1.0817x over previous
import functools

import jax
import jax.numpy as jnp
from jax import lax
from jax.experimental import pallas as pl
from jax.experimental.pallas import tpu as pltpu
from jax.experimental.pallas import tpu_sc as plsc

_N = 10000
_E = 320000
_D = 128
_NC = 2
_NS = 16
_NW = _NC * _NS
_EW = _E // _NW
_C = 120
_NCH = _EW // _C
_TAILE = _EW - _NCH * _C
_RPT = 624
_TAIL = _N - _RPT * _NS


@functools.cache
def _make_agg_kernel():
  mesh = plsc.VectorSubcoreMesh(core_axis_name="c", subcore_axis_name="s",
                                num_cores=_NC, num_subcores=_NS)

  @functools.partial(
      pl.kernel,
      mesh=mesh,
      out_type=jax.ShapeDtypeStruct((_NC, _N, _D), jnp.float32),
      scratch_types=[
          pltpu.VMEM((_EW,), jnp.int32),
          pltpu.VMEM((_EW,), jnp.int32),
          pltpu.VMEM((_C, _D), jnp.float32),
          pltpu.VMEM((_C, _D), jnp.float32),
          pltpu.VMEM_SHARED((_N, _D), jnp.float32),
          pltpu.SemaphoreType.DMA,
          pltpu.SemaphoreType.DMA,
          pltpu.SemaphoreType.DMA,
      ],
  )
  def agg_kernel(x_hbm, src_hbm, dst_hbm, zeros_hbm, out_hbm,
                 src_v, dst_v, rows0, rows1, agg_sh, sem0, sem1, zsem):
    c = lax.axis_index("c")
    s = lax.axis_index("s")
    wid = c * _NS + s
    @pl.when(c == 0)
    def _():
      pltpu.async_copy(
          x_hbm.at[pl.ds(s * _RPT, _RPT)],
          agg_sh.at[pl.ds(s * _RPT, _RPT)], zsem)

    @pl.when(c != 0)
    def _():
      pltpu.async_copy(
          zeros_hbm.at[pl.ds(s * _RPT, _RPT)],
          agg_sh.at[pl.ds(s * _RPT, _RPT)], zsem)

    src_cp = pltpu.async_copy(src_hbm.at[pl.ds(wid * _EW, _EW)], src_v, sem0)
    dst_cp = pltpu.async_copy(dst_hbm.at[pl.ds(wid * _EW, _EW)], dst_v, sem1)
    src_cp.wait()
    dst_cp.wait()
    pltpu.make_async_copy(
        zeros_hbm.at[pl.ds(s * _RPT, _RPT)],
        agg_sh.at[pl.ds(s * _RPT, _RPT)], zsem).wait()

    @pl.when(s == _NS - 1)
    def _():
      @pl.when(c == 0)
      def _():
        pltpu.sync_copy(
            x_hbm.at[pl.ds(_RPT * _NS, _TAIL)],
            agg_sh.at[pl.ds(_RPT * _NS, _TAIL)])

      @pl.when(c != 0)
      def _():
        pltpu.sync_copy(
            zeros_hbm.at[pl.ds(_RPT * _NS, _TAIL)],
            agg_sh.at[pl.ds(_RPT * _NS, _TAIL)])

    plsc.subcore_barrier()

    def src_slice(j):
      return src_v.at[pl.ds(pl.multiple_of(j * _C, 8), _C)]

    def dst_slice(j):
      return dst_v.at[pl.ds(pl.multiple_of(j * _C, 8), _C)]

    pltpu.async_copy(x_hbm.at[src_slice(0)], rows0, sem0)
    pltpu.async_copy(x_hbm.at[src_slice(1)], rows1, sem1)

    def body(i, carry):
      for b, rows, sem in ((0, rows0, sem0), (1, rows1, sem1)):
        j = 2 * i + b
        pltpu.make_async_copy(x_hbm.at[src_slice(j)], rows, sem).wait()
        pltpu.sync_copy(rows, agg_sh.at[dst_slice(j)], add=True)
        nxt = j + 2

        @pl.when(nxt < _NCH)
        def _(rows=rows, sem=sem, nxt=nxt):
          pltpu.async_copy(x_hbm.at[src_slice(nxt)], rows, sem)

      return carry

    lax.fori_loop(0, _NCH // 2, body, 0, unroll=False)
    if _NCH % 2:
      j = _NCH - 1
      pltpu.make_async_copy(x_hbm.at[src_slice(j)], rows0, sem0).wait()
      pltpu.sync_copy(rows0, agg_sh.at[dst_slice(j)], add=True)
    if _TAILE:
      base = _NCH * _C
      pltpu.async_copy(
          x_hbm.at[src_v.at[pl.ds(base, _TAILE)]],
          rows1.at[pl.ds(0, _TAILE)], sem1).wait()
      pltpu.sync_copy(
          rows1.at[pl.ds(0, _TAILE)],
          agg_sh.at[dst_v.at[pl.ds(base, _TAILE)]], add=True)
    plsc.subcore_barrier()
    pltpu.sync_copy(
        agg_sh.at[pl.ds(s * _RPT, _RPT)],
        out_hbm.at[c, pl.ds(s * _RPT, _RPT)])

    @pl.when(s == _NS - 1)
    def _():
      pltpu.sync_copy(
          agg_sh.at[pl.ds(_RPT * _NS, _TAIL)],
          out_hbm.at[c, pl.ds(_RPT * _NS, _TAIL)])

  return agg_kernel


_B = 2000
_K = _N // _B


def _dense_body(last, agg_ref, w1_ref, b1_ref, g_ref, bt_ref,
                w2_ref, b2_ref, og_ref, ob_ref, o_ref,
                t_ref, s1_ref, q1_ref, s2_ref, q2_ref):
  i = pl.program_id(0)

  @pl.when(i == 0)
  def _():
    s1_ref[...] = jnp.zeros_like(s1_ref)
    q1_ref[...] = jnp.zeros_like(q1_ref)
    s2_ref[...] = jnp.zeros_like(s2_ref)
    q2_ref[...] = jnp.zeros_like(q2_ref)

  @pl.when(i < _K)
  def _():
    blk = agg_ref[0] + agg_ref[1]
    t = jnp.dot(blk, w1_ref[...],
                preferred_element_type=jnp.float32) + b1_ref[...]
    t_ref[pl.ds(i * _B, _B), :] = t
    s1_ref[...] += jnp.sum(t, axis=0, keepdims=True)
    q1_ref[...] += jnp.sum(t * t, axis=0, keepdims=True)

  @pl.when((i >= _K) & (i < 2 * _K))
  def _():
    ib = i - _K
    t = t_ref[pl.ds(ib * _B, _B), :]
    m = s1_ref[...] * (1.0 / _N)
    var = q1_ref[...] * (1.0 / _N) - m * m
    r = jnp.maximum((t - m) * (g_ref[...] * lax.rsqrt(var + 1e-5))
                    + bt_ref[...], 0.0)
    v = jnp.dot(r, w2_ref[...],
                preferred_element_type=jnp.float32) + b2_ref[...]
    if last:
      nrm = jnp.sqrt(jnp.sum(v * v, axis=1, keepdims=True))
      o_ref[...] = v / jnp.maximum(nrm, 1e-12)
    else:
      t_ref[pl.ds(ib * _B, _B), :] = v
      s2_ref[...] += jnp.sum(v, axis=0, keepdims=True)
      q2_ref[...] += jnp.sum(v * v, axis=0, keepdims=True)

  if not last:
    @pl.when(i >= 2 * _K)
    def _():
      ic = i - 2 * _K
      v = t_ref[pl.ds(ic * _B, _B), :]
      m2 = s2_ref[...] * (1.0 / _N)
      var2 = q2_ref[...] * (1.0 / _N) - m2 * m2
      o_ref[...] = jnp.maximum(
          (v - m2) * (og_ref[...] * lax.rsqrt(var2 + 1e-5)) + ob_ref[...],
          0.0)


def _dense_layer(agg, w1, b1, g, bt, w2, b2, og, ob, last):
  nsteps = (2 if last else 3) * _K
  ofs = (1 if last else 2) * _K
  full = lambda i: (0, 0)
  return pl.pallas_call(
      functools.partial(_dense_body, last),
      grid=(nsteps,),
      in_specs=[
          pl.BlockSpec((_NC, _B, _D),
                       lambda i: (0, jnp.minimum(i, _K - 1), 0)),
          pl.BlockSpec((_D, _D), full),
          pl.BlockSpec((1, _D), full),
          pl.BlockSpec((1, _D), full),
          pl.BlockSpec((1, _D), full),
          pl.BlockSpec((_D, _D), full),
          pl.BlockSpec((1, _D), full),
          pl.BlockSpec((1, _D), full),
          pl.BlockSpec((1, _D), full),
      ],
      out_specs=pl.BlockSpec((_B, _D),
                             lambda i: (jnp.maximum(i - ofs, 0), 0)),
      out_shape=jax.ShapeDtypeStruct((_N, _D), jnp.float32),
      scratch_shapes=[
          pltpu.VMEM((_N, _D), jnp.float32),
          pltpu.VMEM((1, _D), jnp.float32),
          pltpu.VMEM((1, _D), jnp.float32),
          pltpu.VMEM((1, _D), jnp.float32),
          pltpu.VMEM((1, _D), jnp.float32),
      ],
  )(agg, w1, b1.reshape(1, _D), g.reshape(1, _D), bt.reshape(1, _D),
    w2, b2.reshape(1, _D), og.reshape(1, _D), ob.reshape(1, _D))


def kernel(x, edge_index,
           W1_0, b1_0, g_0, bt_0, W2_0, b2_0,
           W1_1, b1_1, g_1, bt_1, W2_1, b2_1,
           W1_2, b1_2, g_2, bt_2, W2_2, b2_2,
           og_0, ob_0, og_1, ob_1):
  src = edge_index[0]
  dst = edge_index[1]
  zeros = jnp.zeros((_N, _D), jnp.float32)
  params = [
      (W1_0, b1_0, g_0, bt_0, W2_0, b2_0, og_0, ob_0),
      (W1_1, b1_1, g_1, bt_1, W2_1, b2_1, og_1, ob_1),
      (W1_2, b1_2, g_2, bt_2, W2_2, b2_2, og_0, ob_0),
  ]
  for l in range(3):
    w1, b1, g, bt, w2, b2, og, ob = params[l]
    agg = _make_agg_kernel()(x, src, dst, zeros)
    x = _dense_layer(agg, w1, b1, g, bt, w2, b2, og, ob, last=(l == 2))
  return x
